# Initial kernel scaffold; baseline (speedup 1.0000x reference)
import jax
import jax.numpy as jnp
from jax.experimental import pallas as pl


def kernel(x, edge_w, src, dst):
    raise NotImplementedError("write your pallas kernel here")


# speedup vs baseline: 3.4226x; 3.4226x over previous
import functools

import jax
import jax.numpy as jnp
from jax import lax
from jax.experimental import pallas as pl
from jax.experimental.pallas import tpu as pltpu
from jax.experimental.pallas import tpu_sc as plsc

N_CORES = 2
N_SUBCORES = 16
LANES = 16
NW = N_CORES * N_SUBCORES


def _sc_scatter_partials(x, edge_w, src, dst):
    N, D = x.shape
    E = edge_w.shape[0]
    epw = E // NW
    C = 80
    nch = epw // C
    nrc = N // C
    nrc_up = -(-nrc // N_SUBCORES)

    mesh = plsc.VectorSubcoreMesh(
        core_axis_name="c", subcore_axis_name="s",
        num_cores=N_CORES, num_subcores=N_SUBCORES)

    @functools.partial(
        pl.kernel,
        out_type=jax.ShapeDtypeStruct((N_CORES, N, D), jnp.float32),
        mesh=mesh,
        scratch_types=[
            pltpu.VMEM((C,), jnp.int32),
            pltpu.VMEM((C,), jnp.int32),
            pltpu.VMEM((C, D), jnp.float32),
            pltpu.VMEM((C, D), jnp.float32),
            pltpu.VMEM_SHARED((N, D), jnp.float32),
            pltpu.SemaphoreType.DMA,
        ],
    )
    def k(x_hbm, ew_hbm, src_hbm, dst_hbm, out_hbm, sidx, didx, rows, ew,
          acc, sem):
        cid = lax.axis_index("c")
        sid = lax.axis_index("s")
        wid = cid * N_SUBCORES + sid

        zero = jnp.zeros((LANES,), jnp.float32)

        def zero_rows(i, _):
            for j in range(D // LANES):
                rows[i, pl.ds(j * LANES, LANES)] = zero
            return 0

        lax.fori_loop(0, C, zero_rows, 0)

        def zero_acc(b, _):
            chid = b * N_SUBCORES + sid

            @pl.when(chid < nrc)
            def _():
                pltpu.sync_copy(rows, acc.at[pl.ds(chid * C, C)])

            return 0

        lax.fori_loop(0, nrc_up, zero_acc, 0)
        plsc.subcore_barrier()

        def chunk(ch, _):
            base = wid * epw + ch * C
            pltpu.sync_copy(src_hbm.at[pl.ds(base, C)], sidx)
            pltpu.sync_copy(dst_hbm.at[pl.ds(base, C)], didx)
            pltpu.async_copy(x_hbm.at[sidx], rows, sem).wait()
            pltpu.sync_copy(ew_hbm.at[pl.ds(base, C)], ew)

            def mul_row(i, _):
                for j in range(D // LANES):
                    sl = pl.ds(j * LANES, LANES)
                    rows[i, sl] = rows[i, sl] * ew[i, sl]
                return 0

            lax.fori_loop(0, C, mul_row, 0)
            pltpu.sync_copy(rows, acc.at[didx], add=True)
            return 0

        lax.fori_loop(0, nch, chunk, 0)
        plsc.subcore_barrier()

        def writeback(b, _):
            chid = b * N_SUBCORES + sid

            @pl.when(chid < nrc)
            def _():
                pltpu.sync_copy(acc.at[pl.ds(chid * C, C)],
                                out_hbm.at[cid, pl.ds(chid * C, C)])

            return 0

        lax.fori_loop(0, nrc_up, writeback, 0)

    return k(x, edge_w, src, dst)


def _tc_sum_partials(partials, out_dtype):
    _, N, D = partials.shape
    blk = 2000
    grid = N // blk

    def body(p_ref, o_ref):
        o_ref[...] = (p_ref[0] + p_ref[1]).astype(out_dtype)

    return pl.pallas_call(
        body,
        grid=(grid,),
        in_specs=[pl.BlockSpec((2, blk, D), lambda i: (0, i, 0))],
        out_specs=pl.BlockSpec((blk, D), lambda i: (i, 0)),
        out_shape=jax.ShapeDtypeStruct((N, D), out_dtype),
    )(partials)


def kernel(x, edge_w, src, dst):
    src = src.astype(jnp.int32)
    dst = dst.astype(jnp.int32)
    x32 = x.astype(jnp.float32)
    ew32 = edge_w.astype(jnp.float32)
    partials = _sc_scatter_partials(x32, ew32, src, dst)
    return _tc_sum_partials(partials, x.dtype)


# speedup vs baseline: 8.0935x; 2.3647x over previous
import functools

import jax
import jax.numpy as jnp
from jax import lax
from jax.experimental import pallas as pl
from jax.experimental.pallas import tpu as pltpu
from jax.experimental.pallas import tpu_sc as plsc

N_CORES = 2
N_SUBCORES = 16
LANES = 16
NW = N_CORES * N_SUBCORES


def _sc_scatter_partials(x, edge_w, src_r, dst_r, C, SB, nblk):
    N, D = x.shape
    E = edge_w.shape[0]
    epw = E // NW
    nch = SB * nblk
    NB = 2
    nrc = N // C
    nrc_up = -(-nrc // N_SUBCORES)

    mesh = plsc.VectorSubcoreMesh(
        core_axis_name="c", subcore_axis_name="s",
        num_cores=N_CORES, num_subcores=N_SUBCORES)

    @functools.partial(
        pl.kernel,
        out_type=jax.ShapeDtypeStruct((N_CORES, N, D), jnp.float32),
        mesh=mesh,
        scratch_types=[
            pltpu.VMEM((2, SB, C), jnp.int32),
            pltpu.VMEM((2, SB, C), jnp.int32),
            [pltpu.VMEM((C, D), jnp.float32) for _ in range(NB)],
            [pltpu.VMEM((C, D), jnp.float32) for _ in range(NB)],
            [pltpu.VMEM((C, D), jnp.float32) for _ in range(NB)],
            pltpu.VMEM_SHARED((N, D), jnp.float32),
            [pltpu.SemaphoreType.DMA for _ in range(NB)],
            [pltpu.SemaphoreType.DMA for _ in range(NB)],
            [pltpu.SemaphoreType.DMA for _ in range(NB)],
            pltpu.SemaphoreType.DMA,
        ],
    )
    def k(x_hbm, ew_hbm, src_hbm, dst_hbm, out_hbm, sidx, didx, rows, ew,
          msg, acc, sem_g, sem_w, sem_s, sem_i):
        cid = lax.axis_index("c")
        sid = lax.axis_index("s")
        wid = cid * N_SUBCORES + sid

        pltpu.sync_copy(src_hbm.at[wid, 0], sidx.at[0])
        pltpu.sync_copy(dst_hbm.at[wid, 0], didx.at[0])
        if nblk > 1:
            pltpu.async_copy(src_hbm.at[wid, 1], sidx.at[1], sem_i)
            pltpu.async_copy(dst_hbm.at[wid, 1], didx.at[1], sem_i)

        zero = jnp.zeros((LANES,), jnp.float32)

        def zero_rows(i, _):
            for j in range(D // LANES):
                rows[0][i, pl.ds(j * LANES, LANES)] = zero
            return 0

        lax.fori_loop(0, C, zero_rows, 0)

        def zero_acc(b, _):
            chid = b * N_SUBCORES + sid

            @pl.when(chid < nrc)
            def _():
                pltpu.sync_copy(rows[0], acc.at[pl.ds(chid * C, C)])

            return 0

        lax.fori_loop(0, nrc_up, zero_acc, 0)
        plsc.subcore_barrier()

        def issue(ch, b):
            blk = lax.div(ch, SB)
            r = lax.rem(ch, SB)
            p = lax.rem(blk, 2)
            pltpu.async_copy(x_hbm.at[sidx.at[p, r]], rows[b], sem_g[b])
            base = wid * epw + ch * C
            pltpu.async_copy(ew_hbm.at[pl.ds(base, C)], ew[b], sem_w[b])

        def wait_gw(b):
            pltpu.make_async_copy(
                x_hbm.at[sidx.at[0, 0]], rows[b], sem_g[b]).wait()
            pltpu.make_async_copy(
                ew_hbm.at[pl.ds(0, C)], ew[b], sem_w[b]).wait()

        def compute(b):
            def mul_row(i, _):
                for j in range(D // LANES):
                    sl = pl.ds(j * LANES, LANES)
                    msg[b][i, sl] = rows[b][i, sl] * ew[b][i, sl]
                return 0

            lax.fori_loop(0, C, mul_row, 0)

        def scatter(ch, b):
            blk = lax.div(ch, SB)
            r = lax.rem(ch, SB)
            p = lax.rem(blk, 2)
            pltpu.async_copy(msg[b], acc.at[didx.at[p, r]], sem_s[b],
                             add=True)

        def wait_s(b):
            pltpu.make_async_copy(
                msg[b], acc.at[didx.at[0, 0]], sem_s[b]).wait()

        for b in range(NB):
            issue(b, b)
        for b in range(NB):
            wait_gw(b)
            compute(b)
            issue(NB + b, b)
            scatter(b, b)

        def body(g, _):
            for b in range(NB):
                ch = NB + g * NB + b
                blk = lax.div(ch, SB)
                r = lax.rem(ch, SB)
                p = lax.rem(blk, 2)
                wait_gw(b)
                wait_s(b)
                compute(b)

                @pl.when(jnp.logical_and(r == 0, blk + 1 < nblk))
                def _():
                    pltpu.async_copy(src_hbm.at[wid, blk + 1],
                                     sidx.at[1 - p], sem_i)
                    pltpu.async_copy(dst_hbm.at[wid, blk + 1],
                                     didx.at[1 - p], sem_i)

                @pl.when(jnp.logical_and(r == SB - NB, blk + 1 < nblk))
                def _():
                    pltpu.make_async_copy(
                        src_hbm.at[wid, 0], sidx.at[0], sem_i).wait()
                    pltpu.make_async_copy(
                        dst_hbm.at[wid, 0], didx.at[0], sem_i).wait()

                nxt = ch + NB

                @pl.when(nxt < nch)
                def _():
                    issue(nxt, b)

                scatter(ch, b)
            return 0

        lax.fori_loop(0, (nch - NB) // NB, body, 0)
        for b in range(NB):
            wait_s(b)
        plsc.subcore_barrier()

        def writeback(b, _):
            chid = b * N_SUBCORES + sid

            @pl.when(chid < nrc)
            def _():
                pltpu.sync_copy(acc.at[pl.ds(chid * C, C)],
                                out_hbm.at[cid, pl.ds(chid * C, C)])

            return 0

        lax.fori_loop(0, nrc_up, writeback, 0)

    return k(x, edge_w, src_r, dst_r)


def _tc_sum_partials(partials, out_dtype):
    _, N, D = partials.shape
    blk = 2000
    grid = N // blk

    def body(p_ref, o_ref):
        o_ref[...] = (p_ref[0] + p_ref[1]).astype(out_dtype)

    return pl.pallas_call(
        body,
        grid=(grid,),
        in_specs=[pl.BlockSpec((2, blk, D), lambda i: (0, i, 0))],
        out_specs=pl.BlockSpec((blk, D), lambda i: (i, 0)),
        out_shape=jax.ShapeDtypeStruct((N, D), out_dtype),
    )(partials)


def kernel(x, edge_w, src, dst):
    E = edge_w.shape[0]
    C = 40
    SB = 25
    nch = E // NW // C
    nblk = nch // SB
    src_r = src.astype(jnp.int32).reshape(NW, nblk, SB, C)
    dst_r = dst.astype(jnp.int32).reshape(NW, nblk, SB, C)
    x32 = x.astype(jnp.float32)
    ew32 = edge_w.astype(jnp.float32)
    partials = _sc_scatter_partials(x32, ew32, src_r, dst_r, C, SB, nblk)
    return _tc_sum_partials(partials, x.dtype)
